# baseline (speedup 1.0000x reference)
import functools

import jax
import jax.numpy as jnp
import numpy as np
from jax import lax
from jax.experimental import pallas as pl
from jax.experimental.pallas import tpu as pltpu
from jax.experimental.pallas import tpu_sc as plsc

_NC = 2
_NS = 16
_NW = _NC * _NS
_CHUNK = 128


def _softplus(w):
    return jnp.maximum(w, 0.0) + jnp.log1p(jnp.exp(-jnp.abs(w)))


def _sc_gather_msg(P_s, P_r, snd3, rcv3, E):
    D = P_s.shape[1]
    C = snd3.shape[0]
    main = C // _NW
    tail = C - main * _NW
    half = main // 2
    mesh = plsc.VectorSubcoreMesh(core_axis_name="c", subcore_axis_name="s")

    @functools.partial(
        pl.kernel,
        out_type=jax.ShapeDtypeStruct((E, D), jnp.float32),
        mesh=mesh,
        scratch_types=[
            pltpu.VMEM((main, 1, _CHUNK), jnp.int32),
            pltpu.VMEM((main, 1, _CHUNK), jnp.int32),
            pltpu.VMEM((1, 1, _CHUNK), jnp.int32),
            pltpu.VMEM((1, 1, _CHUNK), jnp.int32),
            pltpu.VMEM((_CHUNK, D), jnp.float32),
            pltpu.VMEM((_CHUNK, D), jnp.float32),
            pltpu.VMEM((_CHUNK, D), jnp.float32),
            pltpu.VMEM((_CHUNK, D), jnp.float32),
            pltpu.VMEM((_CHUNK, D), jnp.float32),
            pltpu.VMEM((_CHUNK, D), jnp.float32),
            pltpu.SemaphoreType.DMA,
            pltpu.SemaphoreType.DMA,
            pltpu.SemaphoreType.DMA,
            pltpu.SemaphoreType.DMA,
            pltpu.SemaphoreType.DMA,
            pltpu.SemaphoreType.DMA,
        ],
    )
    def k(ps_hbm, pr_hbm, snd_hbm, rcv_hbm, out_hbm, idx_s, idx_r, idx_ts,
          idx_tr, a0, a1, b0, b1, o0, o1, ga0, ga1, gb0, gb1, w0, w1):
        wid = lax.axis_index("s") * _NC + lax.axis_index("c")
        c0 = wid * main
        A = [a0, a1]
        Bb = [b0, b1]
        O = [o0, o1]
        GA = [ga0, ga1]
        GB = [gb0, gb1]
        W = [w0, w1]

        pltpu.sync_copy(snd_hbm.at[pl.ds(c0, main)], idx_s)
        pltpu.sync_copy(rcv_hbm.at[pl.ds(c0, main)], idx_r)

        def issue(j, b):
            pltpu.async_copy(ps_hbm.at[idx_s.at[j, 0]], A[b], GA[b])
            pltpu.async_copy(pr_hbm.at[idx_r.at[j, 0]], Bb[b], GB[b])

        def add_into(dst, x, y):
            def add_row(r, carry2):
                for l in range(D // 16):
                    sl = pl.ds(l * 16, 16)
                    dst[r, sl] = x[r, sl] + y[r, sl]
                return carry2

            lax.fori_loop(0, _CHUNK, add_row, 0)

        issue(0, 0)
        issue(1, 1)

        def body(jj, carry):
            for b in range(2):
                j = 2 * jj + b
                pltpu.make_async_copy(ps_hbm.at[idx_s.at[0, 0]], A[b],
                                      GA[b]).wait()
                pltpu.make_async_copy(pr_hbm.at[idx_r.at[0, 0]], Bb[b],
                                      GB[b]).wait()

                @pl.when(jj >= 1)
                def _():
                    pltpu.make_async_copy(
                        o0, out_hbm.at[pl.ds(0, _CHUNK), :], W[b]).wait()

                add_into(O[b], A[b], Bb[b])

                @pl.when(jj < half - 1)
                def _():
                    issue(j + 2, b)

                base = (c0 + j) * _CHUNK
                pltpu.async_copy(O[b], out_hbm.at[pl.ds(base, _CHUNK), :], W[b])
            return carry

        lax.fori_loop(0, half, body, 0)
        for b in range(2):
            pltpu.make_async_copy(o0, out_hbm.at[pl.ds(0, _CHUNK), :],
                                  W[b]).wait()

        if tail:
            @pl.when(wid < tail)
            def _():
                ct = C - tail + wid
                pltpu.sync_copy(snd_hbm.at[pl.ds(ct, 1)], idx_ts)
                pltpu.sync_copy(rcv_hbm.at[pl.ds(ct, 1)], idx_tr)
                cp_a = pltpu.async_copy(ps_hbm.at[idx_ts.at[0, 0]], a0, ga0)
                cp_b = pltpu.async_copy(pr_hbm.at[idx_tr.at[0, 0]], b0, gb0)
                cp_a.wait()
                cp_b.wait()
                add_into(o0, a0, b0)
                pltpu.sync_copy(o0, out_hbm.at[pl.ds(ct * _CHUNK, _CHUNK), :])

    return k(P_s, P_r, snd3, rcv3)


def _sc_scatter_add(upd, rcv3, n_nodes):
    E, D = upd.shape
    n_chunks = E // _CHUNK
    per_tile = (n_chunks + _NW - 1) // _NW
    rows_per_sub = ((n_nodes + _NS - 1) // _NS + 7) // 8 * 8
    n_pad = rows_per_sub * _NS
    n_full = rows_per_sub // _CHUNK
    rem = rows_per_sub - n_full * _CHUNK
    mesh = plsc.VectorSubcoreMesh(core_axis_name="c", subcore_axis_name="s")

    C = n_chunks
    main = C // _NW
    tail = C - main * _NW
    half = main // 2

    @functools.partial(
        pl.kernel,
        out_type=jax.ShapeDtypeStruct((_NC * n_pad, D), jnp.float32),
        mesh=mesh,
        scratch_types=[
            pltpu.VMEM((main, 1, _CHUNK), jnp.int32),
            pltpu.VMEM((1, 1, _CHUNK), jnp.int32),
            pltpu.VMEM((_CHUNK, D), jnp.float32),
            pltpu.VMEM((_CHUNK, D), jnp.float32),
            pltpu.VMEM_SHARED((n_pad, D), jnp.float32),
            pltpu.SemaphoreType.DMA,
            pltpu.SemaphoreType.DMA,
        ],
    )
    def k(upd_hbm, rcv_hbm, out_hbm, idx_m, idx_t, a0, a1, acc, r0, r1):
        cid = lax.axis_index("c")
        sid = lax.axis_index("s")
        wid = sid * _NC + cid
        my_row0 = sid * rows_per_sub
        c0 = wid * main
        A = [a0, a1]
        R = [r0, r1]

        def zero_row(r, carry2):
            for l in range(D // 16):
                a0[r, pl.ds(l * 16, 16)] = jnp.zeros((16,), jnp.float32)
            return carry2

        lax.fori_loop(0, _CHUNK, zero_row, 0)
        for p in range(n_full):
            pltpu.sync_copy(a0, acc.at[pl.ds(my_row0 + p * _CHUNK, _CHUNK), :])
        if rem:
            pltpu.sync_copy(a0.at[:rem],
                            acc.at[pl.ds(my_row0 + n_full * _CHUNK, rem), :])
        plsc.subcore_barrier()

        pltpu.sync_copy(rcv_hbm.at[pl.ds(c0, main)], idx_m)

        def issue(j, b):
            base = (c0 + j) * _CHUNK
            pltpu.async_copy(upd_hbm.at[pl.ds(base, _CHUNK), :], A[b], R[b])

        issue(0, 0)
        issue(1, 1)

        def body(jj, carry):
            for b in range(2):
                j = 2 * jj + b
                pltpu.make_async_copy(upd_hbm.at[pl.ds(0, _CHUNK), :], A[b],
                                      R[b]).wait()
                pltpu.sync_copy(A[b], acc.at[idx_m.at[j, 0]], add=True)

                @pl.when(jj < half - 1)
                def _():
                    issue(j + 2, b)

            return carry

        lax.fori_loop(0, half, body, 0)

        if tail:
            @pl.when(wid < tail)
            def _():
                ct = C - tail + wid
                pltpu.sync_copy(rcv_hbm.at[pl.ds(ct, 1)], idx_t)
                pltpu.sync_copy(upd_hbm.at[pl.ds(ct * _CHUNK, _CHUNK), :], a0)
                pltpu.sync_copy(a0, acc.at[idx_t.at[0, 0]], add=True)

        plsc.subcore_barrier()

        out_row0 = cid * n_pad + my_row0
        pltpu.sync_copy(
            acc.at[pl.ds(my_row0, rows_per_sub), :],
            out_hbm.at[pl.ds(out_row0, rows_per_sub), :])

    return k(upd, rcv3)


def _tc_project(nodes, W_es, W_er, blk=2000):
    N, D = nodes.shape
    grid = N // blk

    def body(n_ref, ws_ref, wr_ref, ps_ref, pr_ref):
        x = n_ref[...]
        ps_ref[...] = jnp.dot(x, ws_ref[...], preferred_element_type=jnp.float32)
        pr_ref[...] = jnp.dot(x, wr_ref[...], preferred_element_type=jnp.float32)

    return pl.pallas_call(
        body,
        grid=(grid,),
        in_specs=[
            pl.BlockSpec((blk, D), lambda i: (i, 0)),
            pl.BlockSpec((D, D), lambda i: (0, 0)),
            pl.BlockSpec((D, D), lambda i: (0, 0)),
        ],
        out_specs=[
            pl.BlockSpec((blk, D), lambda i: (i, 0)),
            pl.BlockSpec((blk, D), lambda i: (i, 0)),
        ],
        out_shape=[
            jax.ShapeDtypeStruct((N, D), jnp.float32),
            jax.ShapeDtypeStruct((N, D), jnp.float32),
        ],
    )(nodes, W_es, W_er)


def _tc_edge_update(edges, msg, eg_idx, W_ee, glob, W_eg, b_e, w_res,
                    blk=10000):
    E, D = edges.shape
    Bsz = glob.shape[0]
    grid = E // blk

    def body(e_ref, m_ref, i_ref, wee_ref, g_ref, weg_ref, be_ref, wr_ref,
             upd_ref, eout_ref, agg_ref):
        mult = _softplus(wr_ref[0, 0])
        Ge = jnp.dot(g_ref[...], weg_ref[...],
                     preferred_element_type=jnp.float32) + be_ref[...]
        iot = lax.broadcasted_iota(jnp.int32, (blk, Bsz), 1)
        oh = (i_ref[...] == iot).astype(jnp.float32)
        x = e_ref[...]
        u = jnp.dot(x, wee_ref[...], preferred_element_type=jnp.float32)
        u = u + m_ref[...]
        u = u + jnp.dot(oh, Ge, preferred_element_type=jnp.float32)
        u = jnp.maximum(u, 0.0)
        upd_ref[...] = u
        eout_ref[...] = x + mult * u
        part = lax.dot_general(oh, u, (((0,), (0,)), ((), ())),
                               preferred_element_type=jnp.float32)

        @pl.when(pl.program_id(0) == 0)
        def _():
            agg_ref[...] = jnp.zeros_like(agg_ref)

        agg_ref[...] += part

    return pl.pallas_call(
        body,
        grid=(grid,),
        in_specs=[
            pl.BlockSpec((blk, D), lambda i: (i, 0)),
            pl.BlockSpec((blk, D), lambda i: (i, 0)),
            pl.BlockSpec((blk, 1), lambda i: (i, 0)),
            pl.BlockSpec((D, D), lambda i: (0, 0)),
            pl.BlockSpec((Bsz, D), lambda i: (0, 0)),
            pl.BlockSpec((D, D), lambda i: (0, 0)),
            pl.BlockSpec((1, D), lambda i: (0, 0)),
            pl.BlockSpec((1, 1), lambda i: (0, 0)),
        ],
        out_specs=[
            pl.BlockSpec((blk, D), lambda i: (i, 0)),
            pl.BlockSpec((blk, D), lambda i: (i, 0)),
            pl.BlockSpec((Bsz, D), lambda i: (0, 0)),
        ],
        out_shape=[
            jax.ShapeDtypeStruct((E, D), jnp.float32),
            jax.ShapeDtypeStruct((E, D), jnp.float32),
            jax.ShapeDtypeStruct((Bsz, D), jnp.float32),
        ],
    )(edges, msg, eg_idx, W_ee, glob, W_eg, b_e, w_res)


def _tc_node_update(nodes, adj0, adj1, ng_idx, W_nn, W_na, W_ng,
                    glob, b_n, eagg, W_gn, W_ge, W_gg, b_g, w_res,
                    blk=2000):
    N, D = nodes.shape
    Bsz = glob.shape[0]
    grid = N // blk

    def body(n_ref, a0_ref, a1_ref, i_ref, wnn_ref, wna_ref,
             wng_ref, g_ref, bn_ref, ea1_ref, wgn_ref, wge_ref,
             wgg_ref, bg_ref, wr_ref, nout_ref, agg_ref, gout_ref):
        mult = _softplus(wr_ref[0, 0])
        Gn = jnp.dot(g_ref[...], wng_ref[...],
                     preferred_element_type=jnp.float32) + bn_ref[...]
        iot = lax.broadcasted_iota(jnp.int32, (blk, Bsz), 1)
        oh = (i_ref[...] == iot).astype(jnp.float32)
        x = n_ref[...]
        adj = a0_ref[...] + a1_ref[...]
        v = jnp.dot(x, wnn_ref[...], preferred_element_type=jnp.float32)
        v = v + jnp.dot(adj, wna_ref[...], preferred_element_type=jnp.float32)
        v = v + jnp.dot(oh, Gn, preferred_element_type=jnp.float32)
        v = jnp.maximum(v, 0.0)
        nout_ref[...] = x + mult * v
        part = lax.dot_general(oh, v, (((0,), (0,)), ((), ())),
                               preferred_element_type=jnp.float32)

        @pl.when(pl.program_id(0) == 0)
        def _():
            agg_ref[...] = jnp.zeros_like(agg_ref)
            gout_ref[...] = jnp.zeros_like(gout_ref)

        agg_ref[...] += part

        @pl.when(pl.program_id(0) == grid - 1)
        def _():
            g = g_ref[...]
            u = jnp.dot(agg_ref[...], wgn_ref[...],
                        preferred_element_type=jnp.float32)
            u = u + jnp.dot(ea1_ref[...], wge_ref[...],
                            preferred_element_type=jnp.float32)
            u = u + jnp.dot(g, wgg_ref[...], preferred_element_type=jnp.float32)
            u = jnp.maximum(u + bg_ref[...], 0.0)
            gout_ref[...] = g + mult * u

    return pl.pallas_call(
        body,
        grid=(grid,),
        in_specs=[
            pl.BlockSpec((blk, D), lambda i: (i, 0)),
            pl.BlockSpec((blk, D), lambda i: (i, 0)),
            pl.BlockSpec((blk, D), lambda i: (i, 0)),
            pl.BlockSpec((blk, 1), lambda i: (i, 0)),
            pl.BlockSpec((D, D), lambda i: (0, 0)),
            pl.BlockSpec((D, D), lambda i: (0, 0)),
            pl.BlockSpec((D, D), lambda i: (0, 0)),
            pl.BlockSpec((Bsz, D), lambda i: (0, 0)),
            pl.BlockSpec((1, D), lambda i: (0, 0)),
            pl.BlockSpec((Bsz, D), lambda i: (0, 0)),
            pl.BlockSpec((D, D), lambda i: (0, 0)),
            pl.BlockSpec((D, D), lambda i: (0, 0)),
            pl.BlockSpec((D, D), lambda i: (0, 0)),
            pl.BlockSpec((1, D), lambda i: (0, 0)),
            pl.BlockSpec((1, 1), lambda i: (0, 0)),
        ],
        out_specs=[
            pl.BlockSpec((blk, D), lambda i: (i, 0)),
            pl.BlockSpec((Bsz, D), lambda i: (0, 0)),
            pl.BlockSpec((Bsz, D), lambda i: (0, 0)),
        ],
        out_shape=[
            jax.ShapeDtypeStruct((N, D), jnp.float32),
            jax.ShapeDtypeStruct((Bsz, D), jnp.float32),
            jax.ShapeDtypeStruct((Bsz, D), jnp.float32),
        ],
    )(nodes, adj0, adj1, ng_idx, W_nn, W_na, W_ng, glob, b_n,
      eagg, W_gn, W_ge, W_gg, b_g, w_res)


def kernel(nodes, edges, receivers, senders, global_latent, node_graph_idx,
           edge_graph_idx, W_e, b_e, W_n, b_n, W_g, b_g, w_res):
    N, D = nodes.shape
    E = edges.shape[0]

    W_ee, W_es, W_er, W_eg = (W_e[:D], W_e[D:2 * D], W_e[2 * D:3 * D],
                              W_e[3 * D:])
    W_nn, W_na, W_ng = W_n[:D], W_n[D:2 * D], W_n[2 * D:]
    W_gn, W_ge, W_gg = W_g[:D], W_g[D:2 * D], W_g[2 * D:]
    b_e2 = b_e.reshape(1, D)
    b_n2 = b_n.reshape(1, D)
    b_g2 = b_g.reshape(1, D)
    wr2 = w_res.reshape(1, 1)

    C = E // _CHUNK
    snd3 = senders.reshape(C, 1, _CHUNK)
    rcv3 = receivers.reshape(C, 1, _CHUNK)

    P_s, P_r = _tc_project(nodes, W_es, W_er)
    msg = _sc_gather_msg(P_s, P_r, snd3, rcv3, E)
    upd, edges_out, eagg = _tc_edge_update(
        edges, msg, edge_graph_idx.reshape(E, 1), W_ee, global_latent,
        W_eg, b_e2, wr2)
    adj2 = _sc_scatter_add(upd, rcv3, N)
    n_pad = adj2.shape[0] // 2
    nodes_out, _, global_out = _tc_node_update(
        nodes, adj2[:N], adj2[n_pad:n_pad + N],
        node_graph_idx.reshape(N, 1), W_nn, W_na,
        W_ng, global_latent, b_n2, eagg, W_gn, W_ge, W_gg, b_g2, wr2)
    return (nodes_out, edges_out, global_out)


# speedup vs baseline: 1.0025x; 1.0025x over previous
import functools

import jax
import jax.numpy as jnp
import numpy as np
from jax import lax
from jax.experimental import pallas as pl
from jax.experimental.pallas import tpu as pltpu
from jax.experimental.pallas import tpu_sc as plsc

_NC = 2
_NS = 16
_NW = _NC * _NS
_CHUNK = 128


def _softplus(w):
    return jnp.maximum(w, 0.0) + jnp.log1p(jnp.exp(-jnp.abs(w)))


def _sc_gather_msg(P_s, P_r, snd3, rcv3, E):
    D = P_s.shape[1]
    C = snd3.shape[0]
    main = C // _NW
    tail = C - main * _NW
    half = main // 2
    mesh = plsc.VectorSubcoreMesh(core_axis_name="c", subcore_axis_name="s")

    @functools.partial(
        pl.kernel,
        out_type=jax.ShapeDtypeStruct((E, D), jnp.float32),
        mesh=mesh,
        scratch_types=[
            pltpu.VMEM((main, 1, _CHUNK), jnp.int32),
            pltpu.VMEM((main, 1, _CHUNK), jnp.int32),
            pltpu.VMEM((1, 1, _CHUNK), jnp.int32),
            pltpu.VMEM((1, 1, _CHUNK), jnp.int32),
            pltpu.VMEM((_CHUNK, D), jnp.float32),
            pltpu.VMEM((_CHUNK, D), jnp.float32),
            pltpu.VMEM((_CHUNK, D), jnp.float32),
            pltpu.VMEM((_CHUNK, D), jnp.float32),
            pltpu.VMEM((_CHUNK, D), jnp.float32),
            pltpu.VMEM((_CHUNK, D), jnp.float32),
            pltpu.SemaphoreType.DMA,
            pltpu.SemaphoreType.DMA,
            pltpu.SemaphoreType.DMA,
            pltpu.SemaphoreType.DMA,
            pltpu.SemaphoreType.DMA,
            pltpu.SemaphoreType.DMA,
        ],
    )
    def k(ps_hbm, pr_hbm, snd_hbm, rcv_hbm, out_hbm, idx_s, idx_r, idx_ts,
          idx_tr, a0, a1, b0, b1, o0, o1, ga0, ga1, gb0, gb1, w0, w1):
        wid = lax.axis_index("s") * _NC + lax.axis_index("c")
        c0 = wid * main
        A = [a0, a1]
        Bb = [b0, b1]
        O = [o0, o1]
        GA = [ga0, ga1]
        GB = [gb0, gb1]
        W = [w0, w1]

        pltpu.sync_copy(snd_hbm.at[pl.ds(c0, main)], idx_s)
        pltpu.sync_copy(rcv_hbm.at[pl.ds(c0, main)], idx_r)

        def issue(j, b):
            pltpu.async_copy(ps_hbm.at[idx_s.at[j, 0]], A[b], GA[b])
            pltpu.async_copy(pr_hbm.at[idx_r.at[j, 0]], Bb[b], GB[b])

        def add_into(dst, x, y):
            def add_row(r, carry2):
                for l in range(D // 16):
                    sl = pl.ds(l * 16, 16)
                    dst[r, sl] = x[r, sl] + y[r, sl]
                return carry2

            lax.fori_loop(0, _CHUNK, add_row, 0)

        issue(0, 0)
        issue(1, 1)

        def body(jj, carry):
            for b in range(2):
                j = 2 * jj + b
                pltpu.make_async_copy(ps_hbm.at[idx_s.at[0, 0]], A[b],
                                      GA[b]).wait()
                pltpu.make_async_copy(pr_hbm.at[idx_r.at[0, 0]], Bb[b],
                                      GB[b]).wait()

                @pl.when(jj >= 1)
                def _():
                    pltpu.make_async_copy(
                        o0, out_hbm.at[pl.ds(0, _CHUNK), :], W[b]).wait()

                add_into(O[b], A[b], Bb[b])

                @pl.when(jj < half - 1)
                def _():
                    issue(j + 2, b)

                base = (c0 + j) * _CHUNK
                pltpu.async_copy(O[b], out_hbm.at[pl.ds(base, _CHUNK), :], W[b])
            return carry

        lax.fori_loop(0, half, body, 0)
        for b in range(2):
            pltpu.make_async_copy(o0, out_hbm.at[pl.ds(0, _CHUNK), :],
                                  W[b]).wait()

        if tail:
            @pl.when(wid < tail)
            def _():
                ct = C - tail + wid
                pltpu.sync_copy(snd_hbm.at[pl.ds(ct, 1)], idx_ts)
                pltpu.sync_copy(rcv_hbm.at[pl.ds(ct, 1)], idx_tr)
                cp_a = pltpu.async_copy(ps_hbm.at[idx_ts.at[0, 0]], a0, ga0)
                cp_b = pltpu.async_copy(pr_hbm.at[idx_tr.at[0, 0]], b0, gb0)
                cp_a.wait()
                cp_b.wait()
                add_into(o0, a0, b0)
                pltpu.sync_copy(o0, out_hbm.at[pl.ds(ct * _CHUNK, _CHUNK), :])

    return k(P_s, P_r, snd3, rcv3)


def _sc_scatter_add(upd, rcv3, n_nodes):
    E, D = upd.shape
    n_chunks = E // _CHUNK
    per_tile = (n_chunks + _NW - 1) // _NW
    rows_per_sub = ((n_nodes + _NS - 1) // _NS + 7) // 8 * 8
    n_pad = rows_per_sub * _NS
    n_full = rows_per_sub // _CHUNK
    rem = rows_per_sub - n_full * _CHUNK
    mesh = plsc.VectorSubcoreMesh(core_axis_name="c", subcore_axis_name="s")

    C = n_chunks
    main = C // _NW
    tail = C - main * _NW
    half = main // 2

    @functools.partial(
        pl.kernel,
        out_type=jax.ShapeDtypeStruct((_NC * n_pad, D), jnp.float32),
        mesh=mesh,
        scratch_types=[
            pltpu.VMEM((main, 1, _CHUNK), jnp.int32),
            pltpu.VMEM((1, 1, _CHUNK), jnp.int32),
            pltpu.VMEM((_CHUNK, D), jnp.float32),
            pltpu.VMEM((_CHUNK, D), jnp.float32),
            pltpu.VMEM_SHARED((n_pad, D), jnp.float32),
            pltpu.SemaphoreType.DMA,
            pltpu.SemaphoreType.DMA,
        ],
    )
    def k(upd_hbm, rcv_hbm, out_hbm, idx_m, idx_t, a0, a1, acc, r0, r1):
        cid = lax.axis_index("c")
        sid = lax.axis_index("s")
        wid = sid * _NC + cid
        my_row0 = sid * rows_per_sub
        c0 = wid * main
        A = [a0, a1]
        R = [r0, r1]

        def zero_row(r, carry2):
            for l in range(D // 16):
                a0[r, pl.ds(l * 16, 16)] = jnp.zeros((16,), jnp.float32)
            return carry2

        lax.fori_loop(0, _CHUNK, zero_row, 0)
        for p in range(n_full):
            pltpu.sync_copy(a0, acc.at[pl.ds(my_row0 + p * _CHUNK, _CHUNK), :])
        if rem:
            pltpu.sync_copy(a0.at[:rem],
                            acc.at[pl.ds(my_row0 + n_full * _CHUNK, rem), :])
        plsc.subcore_barrier()

        pltpu.sync_copy(rcv_hbm.at[pl.ds(c0, main)], idx_m)

        def issue(j, b):
            base = (c0 + j) * _CHUNK
            pltpu.async_copy(upd_hbm.at[pl.ds(base, _CHUNK), :], A[b], R[b])

        issue(0, 0)
        issue(1, 1)

        def body(jj, carry):
            for b in range(2):
                j = 2 * jj + b
                pltpu.make_async_copy(upd_hbm.at[pl.ds(0, _CHUNK), :], A[b],
                                      R[b]).wait()
                pltpu.sync_copy(A[b], acc.at[idx_m.at[j, 0]], add=True)

                @pl.when(jj < half - 1)
                def _():
                    issue(j + 2, b)

            return carry

        lax.fori_loop(0, half, body, 0)

        if tail:
            @pl.when(wid < tail)
            def _():
                ct = C - tail + wid
                pltpu.sync_copy(rcv_hbm.at[pl.ds(ct, 1)], idx_t)
                pltpu.sync_copy(upd_hbm.at[pl.ds(ct * _CHUNK, _CHUNK), :], a0)
                pltpu.sync_copy(a0, acc.at[idx_t.at[0, 0]], add=True)

        plsc.subcore_barrier()

        out_row0 = cid * n_pad + my_row0
        pltpu.sync_copy(
            acc.at[pl.ds(my_row0, rows_per_sub), :],
            out_hbm.at[pl.ds(out_row0, rows_per_sub), :])

    return k(upd, rcv3)


def _tc_project(nodes, W_es, W_er, blk=2000):
    N, D = nodes.shape
    grid = N // blk

    def body(n_ref, ws_ref, wr_ref, ps_ref, pr_ref):
        x = n_ref[...]
        ps_ref[...] = jnp.dot(x, ws_ref[...], preferred_element_type=jnp.float32)
        pr_ref[...] = jnp.dot(x, wr_ref[...], preferred_element_type=jnp.float32)

    return pl.pallas_call(
        body,
        grid=(grid,),
        in_specs=[
            pl.BlockSpec((blk, D), lambda i: (i, 0)),
            pl.BlockSpec((D, D), lambda i: (0, 0)),
            pl.BlockSpec((D, D), lambda i: (0, 0)),
        ],
        out_specs=[
            pl.BlockSpec((blk, D), lambda i: (i, 0)),
            pl.BlockSpec((blk, D), lambda i: (i, 0)),
        ],
        out_shape=[
            jax.ShapeDtypeStruct((N, D), jnp.float32),
            jax.ShapeDtypeStruct((N, D), jnp.float32),
        ],
    )(nodes, W_es, W_er)


def _tc_edge_update(edges, msg, eg_idx, W_ee, glob, W_eg, b_e, w_res,
                    blk=8000):
    E, D = edges.shape
    Bsz = glob.shape[0]
    grid = E // blk

    def body(e_ref, m_ref, i_ref, wee_ref, g_ref, weg_ref, be_ref, wr_ref,
             upd_ref, eout_ref, agg_ref):
        mult = _softplus(wr_ref[0, 0])
        Ge = jnp.dot(g_ref[...], weg_ref[...],
                     preferred_element_type=jnp.float32) + be_ref[...]
        iot = lax.broadcasted_iota(jnp.int32, (blk, Bsz), 1)
        oh = (i_ref[...] == iot).astype(jnp.float32)
        x = e_ref[...]
        u = jnp.dot(x, wee_ref[...], preferred_element_type=jnp.float32)
        u = u + m_ref[...]
        u = u + jnp.dot(oh, Ge, preferred_element_type=jnp.float32)
        u = jnp.maximum(u, 0.0)
        upd_ref[...] = u
        eout_ref[...] = x + mult * u
        part = lax.dot_general(oh, u, (((0,), (0,)), ((), ())),
                               preferred_element_type=jnp.float32)

        @pl.when(pl.program_id(0) == 0)
        def _():
            agg_ref[...] = jnp.zeros_like(agg_ref)

        agg_ref[...] += part

    return pl.pallas_call(
        body,
        grid=(grid,),
        in_specs=[
            pl.BlockSpec((blk, D), lambda i: (i, 0)),
            pl.BlockSpec((blk, D), lambda i: (i, 0)),
            pl.BlockSpec((blk, 1), lambda i: (i, 0)),
            pl.BlockSpec((D, D), lambda i: (0, 0)),
            pl.BlockSpec((Bsz, D), lambda i: (0, 0)),
            pl.BlockSpec((D, D), lambda i: (0, 0)),
            pl.BlockSpec((1, D), lambda i: (0, 0)),
            pl.BlockSpec((1, 1), lambda i: (0, 0)),
        ],
        out_specs=[
            pl.BlockSpec((blk, D), lambda i: (i, 0)),
            pl.BlockSpec((blk, D), lambda i: (i, 0)),
            pl.BlockSpec((Bsz, D), lambda i: (0, 0)),
        ],
        out_shape=[
            jax.ShapeDtypeStruct((E, D), jnp.float32),
            jax.ShapeDtypeStruct((E, D), jnp.float32),
            jax.ShapeDtypeStruct((Bsz, D), jnp.float32),
        ],
    )(edges, msg, eg_idx, W_ee, glob, W_eg, b_e, w_res)


def _tc_node_update(nodes, adj0, adj1, ng_idx, W_nn, W_na, W_ng,
                    glob, b_n, eagg, W_gn, W_ge, W_gg, b_g, w_res,
                    blk=2000):
    N, D = nodes.shape
    Bsz = glob.shape[0]
    grid = N // blk

    def body(n_ref, a0_ref, a1_ref, i_ref, wnn_ref, wna_ref,
             wng_ref, g_ref, bn_ref, ea1_ref, wgn_ref, wge_ref,
             wgg_ref, bg_ref, wr_ref, nout_ref, agg_ref, gout_ref):
        mult = _softplus(wr_ref[0, 0])
        Gn = jnp.dot(g_ref[...], wng_ref[...],
                     preferred_element_type=jnp.float32) + bn_ref[...]
        iot = lax.broadcasted_iota(jnp.int32, (blk, Bsz), 1)
        oh = (i_ref[...] == iot).astype(jnp.float32)
        x = n_ref[...]
        adj = a0_ref[...] + a1_ref[...]
        v = jnp.dot(x, wnn_ref[...], preferred_element_type=jnp.float32)
        v = v + jnp.dot(adj, wna_ref[...], preferred_element_type=jnp.float32)
        v = v + jnp.dot(oh, Gn, preferred_element_type=jnp.float32)
        v = jnp.maximum(v, 0.0)
        nout_ref[...] = x + mult * v
        part = lax.dot_general(oh, v, (((0,), (0,)), ((), ())),
                               preferred_element_type=jnp.float32)

        @pl.when(pl.program_id(0) == 0)
        def _():
            agg_ref[...] = jnp.zeros_like(agg_ref)
            gout_ref[...] = jnp.zeros_like(gout_ref)

        agg_ref[...] += part

        @pl.when(pl.program_id(0) == grid - 1)
        def _():
            g = g_ref[...]
            u = jnp.dot(agg_ref[...], wgn_ref[...],
                        preferred_element_type=jnp.float32)
            u = u + jnp.dot(ea1_ref[...], wge_ref[...],
                            preferred_element_type=jnp.float32)
            u = u + jnp.dot(g, wgg_ref[...], preferred_element_type=jnp.float32)
            u = jnp.maximum(u + bg_ref[...], 0.0)
            gout_ref[...] = g + mult * u

    return pl.pallas_call(
        body,
        grid=(grid,),
        in_specs=[
            pl.BlockSpec((blk, D), lambda i: (i, 0)),
            pl.BlockSpec((blk, D), lambda i: (i, 0)),
            pl.BlockSpec((blk, D), lambda i: (i, 0)),
            pl.BlockSpec((blk, 1), lambda i: (i, 0)),
            pl.BlockSpec((D, D), lambda i: (0, 0)),
            pl.BlockSpec((D, D), lambda i: (0, 0)),
            pl.BlockSpec((D, D), lambda i: (0, 0)),
            pl.BlockSpec((Bsz, D), lambda i: (0, 0)),
            pl.BlockSpec((1, D), lambda i: (0, 0)),
            pl.BlockSpec((Bsz, D), lambda i: (0, 0)),
            pl.BlockSpec((D, D), lambda i: (0, 0)),
            pl.BlockSpec((D, D), lambda i: (0, 0)),
            pl.BlockSpec((D, D), lambda i: (0, 0)),
            pl.BlockSpec((1, D), lambda i: (0, 0)),
            pl.BlockSpec((1, 1), lambda i: (0, 0)),
        ],
        out_specs=[
            pl.BlockSpec((blk, D), lambda i: (i, 0)),
            pl.BlockSpec((Bsz, D), lambda i: (0, 0)),
            pl.BlockSpec((Bsz, D), lambda i: (0, 0)),
        ],
        out_shape=[
            jax.ShapeDtypeStruct((N, D), jnp.float32),
            jax.ShapeDtypeStruct((Bsz, D), jnp.float32),
            jax.ShapeDtypeStruct((Bsz, D), jnp.float32),
        ],
    )(nodes, adj0, adj1, ng_idx, W_nn, W_na, W_ng, glob, b_n,
      eagg, W_gn, W_ge, W_gg, b_g, w_res)


def kernel(nodes, edges, receivers, senders, global_latent, node_graph_idx,
           edge_graph_idx, W_e, b_e, W_n, b_n, W_g, b_g, w_res):
    N, D = nodes.shape
    E = edges.shape[0]

    W_ee, W_es, W_er, W_eg = (W_e[:D], W_e[D:2 * D], W_e[2 * D:3 * D],
                              W_e[3 * D:])
    W_nn, W_na, W_ng = W_n[:D], W_n[D:2 * D], W_n[2 * D:]
    W_gn, W_ge, W_gg = W_g[:D], W_g[D:2 * D], W_g[2 * D:]
    b_e2 = b_e.reshape(1, D)
    b_n2 = b_n.reshape(1, D)
    b_g2 = b_g.reshape(1, D)
    wr2 = w_res.reshape(1, 1)

    C = E // _CHUNK
    snd3 = senders.reshape(C, 1, _CHUNK)
    rcv3 = receivers.reshape(C, 1, _CHUNK)

    P_s, P_r = _tc_project(nodes, W_es, W_er)
    msg = _sc_gather_msg(P_s, P_r, snd3, rcv3, E)
    upd, edges_out, eagg = _tc_edge_update(
        edges, msg, edge_graph_idx.reshape(E, 1), W_ee, global_latent,
        W_eg, b_e2, wr2)
    adj2 = _sc_scatter_add(upd, rcv3, N)
    n_pad = adj2.shape[0] // 2
    nodes_out, _, global_out = _tc_node_update(
        nodes, adj2[:N], adj2[n_pad:n_pad + N],
        node_graph_idx.reshape(N, 1), W_nn, W_na,
        W_ng, global_latent, b_n2, eagg, W_gn, W_ge, W_gg, b_g2, wr2)
    return (nodes_out, edges_out, global_out)
